# baseline (device time: 315295 ns/iter reference)
import jax
import jax.numpy as jnp
from jax import lax
from jax.experimental import pallas as pl
from jax.experimental.pallas import tpu as pltpu

N_DEV = 4
N_HOPS = N_DEV - 1
N_SUB = 2
PLUS, MINUS = 0, 1


def kernel(x):
    m, n = x.shape
    half = m // 2
    ch = half // N_DEV
    ch2 = ch // N_SUB

    def body(x_ref, out_ref, comm_ref, send_sems, recv_sems, copy_sems):
        my = lax.axis_index("i")
        left = (my + N_DEV - 1) % N_DEV
        right = (my + 1) % N_DEV

        peer_out = {PLUS: right, MINUS: left}
        peer_in = {PLUS: left, MINUS: right}

        def up(dirn, k):
            if dirn == PLUS:
                return (my - k + N_DEV) % N_DEV
            return (my + k) % N_DEV

        def rows(dirn, c, t):
            return pl.ds(dirn * half + c * ch + t * ch2, ch2)

        def load_chunk(dirn, k):
            c = up(dirn, k)
            rws = pl.ds(dirn * half + c * ch, ch)
            cp = pltpu.make_async_copy(
                x_ref.at[rws], out_ref.at[rws], copy_sems.at[dirn, k]
            )
            cp.start()
            return cp

        loads = {
            (d, k): load_chunk(d, k)
            for d in (PLUS, MINUS)
            for k in range(N_DEV)
        }

        barrier_sem = pltpu.get_barrier_semaphore()
        for nbr in (left, right):
            pl.semaphore_signal(
                barrier_sem, inc=1,
                device_id=(nbr,), device_id_type=pl.DeviceIdType.MESH,
            )
        pl.semaphore_wait(barrier_sem, 2)

        loads[PLUS, 0].wait()
        loads[MINUS, 0].wait()

        def rchunk(dirn, h):
            return up(dirn, h + 1 if h < N_HOPS else h - N_HOPS)

        def schunk(dirn, h):
            return my if h == 0 else rchunk(dirn, h - 1)

        def mk(dirn, h, t, is_send):
            c = schunk(dirn, h) if is_send else rchunk(dirn, h)
            if h < N_HOPS:
                buf = comm_ref.at[dirn, h, t]
            else:
                buf = out_ref.at[rows(dirn, c, t), :]
            return pltpu.make_async_remote_copy(
                src_ref=out_ref.at[rows(dirn, c, t), :],
                dst_ref=buf,
                send_sem=send_sems.at[dirn, h, t],
                recv_sem=recv_sems.at[dirn, h, t],
                device_id=(peer_out[dirn] if is_send else peer_in[dirn],),
                device_id_type=pl.DeviceIdType.MESH,
            )

        for t in range(N_SUB):
            for d in (PLUS, MINUS):
                mk(d, 0, t, True).start()

        for h in range(2 * N_HOPS):
            if h < N_HOPS:
                for d in (PLUS, MINUS):
                    loads[d, h + 1].wait()
            for t in range(N_SUB):
                for d in (PLUS, MINUS):
                    mk(d, h, t, False).wait_recv()
                    if h < N_HOPS:
                        out_ref[rows(d, rchunk(d, h), t), :] += (
                            comm_ref[d, h, t]
                        )
                    if h + 1 < 2 * N_HOPS:
                        mk(d, h + 1, t, True).start()

        for h in range(2 * N_HOPS):
            for t in range(N_SUB):
                for d in (PLUS, MINUS):
                    mk(d, h, t, True).wait_send()

    return pl.pallas_call(
        body,
        out_shape=jax.ShapeDtypeStruct((m, n), x.dtype),
        in_specs=[pl.BlockSpec(memory_space=pl.ANY)],
        out_specs=pl.BlockSpec(memory_space=pltpu.VMEM),
        scratch_shapes=[
            pltpu.VMEM((2, N_HOPS, N_SUB, ch2, n), x.dtype),
            pltpu.SemaphoreType.DMA((2, 2 * N_HOPS, N_SUB)),
            pltpu.SemaphoreType.DMA((2, 2 * N_HOPS, N_SUB)),
            pltpu.SemaphoreType.DMA((2, N_DEV)),
        ],
        compiler_params=pltpu.CompilerParams(
            collective_id=0,
            vmem_limit_bytes=63 * 1024 * 1024,
        ),
    )(x)


# device time: 311609 ns/iter; 1.0118x vs baseline; 1.0118x over previous
import jax
import jax.numpy as jnp
from jax import lax
from jax.experimental import pallas as pl
from jax.experimental.pallas import tpu as pltpu

N_DEV = 4
N_HOPS = N_DEV - 1
N_SUB = 2
PLUS, MINUS = 0, 1


def kernel(x):
    m, n = x.shape
    half = m // 2
    ch = half // N_DEV
    ch2 = ch // N_SUB

    def body(x_ref, out_ref, comm_ref, send_sems, recv_sems, copy_sems):
        my = lax.axis_index("i")
        left = (my + N_DEV - 1) % N_DEV
        right = (my + 1) % N_DEV

        peer_out = {PLUS: right, MINUS: left}
        peer_in = {PLUS: left, MINUS: right}

        def up(dirn, k):
            if dirn == PLUS:
                return (my - k + N_DEV) % N_DEV
            return (my + k) % N_DEV

        def rows(dirn, c, t):
            return pl.ds(dirn * half + c * ch + t * ch2, ch2)

        def load_chunk(dirn, k):
            c = up(dirn, k)
            rws = pl.ds(dirn * half + c * ch, ch)
            cp = pltpu.make_async_copy(
                x_ref.at[rws], out_ref.at[rws], copy_sems.at[dirn, k]
            )
            cp.start()
            return cp

        loads = {
            (d, k): load_chunk(d, k)
            for k in range(N_DEV)
            for d in (PLUS, MINUS)
        }

        barrier_sem = pltpu.get_barrier_semaphore()
        for nbr in (left, right):
            pl.semaphore_signal(
                barrier_sem, inc=1,
                device_id=(nbr,), device_id_type=pl.DeviceIdType.MESH,
            )
        pl.semaphore_wait(barrier_sem, 2)

        loads[PLUS, 0].wait()
        loads[MINUS, 0].wait()

        def rchunk(dirn, h):
            return up(dirn, h + 1 if h < N_HOPS else h - N_HOPS)

        def schunk(dirn, h):
            return my if h == 0 else rchunk(dirn, h - 1)

        def mk(dirn, h, t, is_send):
            c = schunk(dirn, h) if is_send else rchunk(dirn, h)
            if h < N_HOPS:
                buf = comm_ref.at[dirn, h, t]
            else:
                buf = out_ref.at[rows(dirn, c, t), :]
            return pltpu.make_async_remote_copy(
                src_ref=out_ref.at[rows(dirn, c, t), :],
                dst_ref=buf,
                send_sem=send_sems.at[dirn, h, t],
                recv_sem=recv_sems.at[dirn, h, t],
                device_id=(peer_out[dirn] if is_send else peer_in[dirn],),
                device_id_type=pl.DeviceIdType.MESH,
            )

        for t in range(N_SUB):
            for d in (PLUS, MINUS):
                mk(d, 0, t, True).start()

        for h in range(2 * N_HOPS):
            if h < N_HOPS:
                for d in (PLUS, MINUS):
                    loads[d, h + 1].wait()
            for t in range(N_SUB):
                for d in (PLUS, MINUS):
                    mk(d, h, t, False).wait_recv()
                    if h < N_HOPS:
                        out_ref[rows(d, rchunk(d, h), t), :] += (
                            comm_ref[d, h, t]
                        )
                    if h + 1 < 2 * N_HOPS:
                        mk(d, h + 1, t, True).start()

        for h in range(2 * N_HOPS):
            for t in range(N_SUB):
                for d in (PLUS, MINUS):
                    mk(d, h, t, True).wait_send()

    return pl.pallas_call(
        body,
        out_shape=jax.ShapeDtypeStruct((m, n), x.dtype),
        in_specs=[pl.BlockSpec(memory_space=pl.ANY)],
        out_specs=pl.BlockSpec(memory_space=pltpu.VMEM),
        scratch_shapes=[
            pltpu.VMEM((2, N_HOPS, N_SUB, ch2, n), x.dtype),
            pltpu.SemaphoreType.DMA((2, 2 * N_HOPS, N_SUB)),
            pltpu.SemaphoreType.DMA((2, 2 * N_HOPS, N_SUB)),
            pltpu.SemaphoreType.DMA((2, N_DEV)),
        ],
        compiler_params=pltpu.CompilerParams(
            collective_id=0,
            vmem_limit_bytes=63 * 1024 * 1024,
        ),
    )(x)


# device time: 302702 ns/iter; 1.0416x vs baseline; 1.0294x over previous
import contextlib
import os

import jax
import jax.numpy as jnp
from jax import lax
from jax.experimental import pallas as pl
from jax.experimental.pallas import tpu as pltpu

_PROFILE = os.environ.get("AR_PROFILE") == "1"


def _scope(name):
    return jax.named_scope(name) if _PROFILE else contextlib.nullcontext()


N_DEV = 4
N_HOPS = N_DEV - 1
N_SUB = 2
PLUS, MINUS = 0, 1


def kernel(x):
    m, n = x.shape
    half = m // 2
    ch = half // N_DEV
    ch2 = ch // N_SUB

    def body(
        x_ref, out_ref, acc_ref, comm_ref,
        send_sems, recv_sems, copy_sems, store_sems,
    ):
        my = lax.axis_index("i")
        left = (my + N_DEV - 1) % N_DEV
        right = (my + 1) % N_DEV

        peer_out = {PLUS: right, MINUS: left}
        peer_in = {PLUS: left, MINUS: right}

        def up(dirn, k):
            if dirn == PLUS:
                return (my - k + N_DEV) % N_DEV
            return (my + k) % N_DEV

        def rows(dirn, c, t):
            return pl.ds(dirn * half + c * ch + t * ch2, ch2)

        def load_chunk(dirn, k):
            c = up(dirn, k)
            rws = pl.ds(dirn * half + c * ch, ch)
            cp = pltpu.make_async_copy(
                x_ref.at[rws], acc_ref.at[rws], copy_sems.at[dirn, k]
            )
            cp.start()
            return cp

        loads = {
            (d, k): load_chunk(d, k)
            for k in range(N_DEV)
            for d in (PLUS, MINUS)
        }

        def store(dirn, k, t):
            rws = rows(dirn, up(dirn, k), t)
            cp = pltpu.make_async_copy(
                acc_ref.at[rws], out_ref.at[rws], store_sems.at[dirn, k, t]
            )
            cp.start()
            return cp

        with _scope("barrier"):
            barrier_sem = pltpu.get_barrier_semaphore()
            for nbr in (left, right):
                pl.semaphore_signal(
                    barrier_sem, inc=1,
                    device_id=(nbr,), device_id_type=pl.DeviceIdType.MESH,
                )
            pl.semaphore_wait(barrier_sem, 2)

        with _scope("load0"):
            loads[PLUS, 0].wait()
            loads[MINUS, 0].wait()

        def rchunk(dirn, h):
            return up(dirn, h + 1 if h < N_HOPS else h - N_HOPS)

        def schunk(dirn, h):
            return my if h == 0 else rchunk(dirn, h - 1)

        def mk(dirn, h, t, is_send):
            c = schunk(dirn, h) if is_send else rchunk(dirn, h)
            if h < N_HOPS:
                buf = comm_ref.at[dirn, h, t]
            else:
                buf = acc_ref.at[rows(dirn, c, t), :]
            return pltpu.make_async_remote_copy(
                src_ref=acc_ref.at[rows(dirn, c, t), :],
                dst_ref=buf,
                send_sem=send_sems.at[dirn, h, t],
                recv_sem=recv_sems.at[dirn, h, t],
                device_id=(peer_out[dirn] if is_send else peer_in[dirn],),
                device_id_type=pl.DeviceIdType.MESH,
            )

        with _scope("send0"):
            for t in range(N_SUB):
                for d in (PLUS, MINUS):
                    mk(d, 0, t, True).start()

        for h in range(2 * N_HOPS):
            with _scope(f"hop#h={h}"):
                if h < N_HOPS:
                    for d in (PLUS, MINUS):
                        loads[d, h + 1].wait()
                for t in range(N_SUB):
                    for d in (PLUS, MINUS):
                        mk(d, h, t, False).wait_recv()
                        if h < N_HOPS:
                            acc_ref[rows(d, rchunk(d, h), t), :] += (
                                comm_ref[d, h, t]
                            )
                            if h == N_HOPS - 1:
                                store(d, N_DEV - 1, t)
                        else:
                            store(d, h - N_HOPS, t)
                        if h + 1 < 2 * N_HOPS:
                            mk(d, h + 1, t, True).start()

        with _scope("drain"):
            for d in (PLUS, MINUS):
                for k in range(N_DEV):
                    for t in range(N_SUB):
                        rws = rows(d, up(d, k), t)
                        pltpu.make_async_copy(
                            acc_ref.at[rws], out_ref.at[rws],
                            store_sems.at[d, k, t],
                        ).wait()
            for h in range(2 * N_HOPS):
                for t in range(N_SUB):
                    for d in (PLUS, MINUS):
                        mk(d, h, t, True).wait_send()

    return pl.pallas_call(
        body,
        out_shape=jax.ShapeDtypeStruct((m, n), x.dtype),
        in_specs=[pl.BlockSpec(memory_space=pl.ANY)],
        out_specs=pl.BlockSpec(memory_space=pl.ANY),
        scratch_shapes=[
            pltpu.VMEM((m, n), x.dtype),
            pltpu.VMEM((2, N_HOPS, N_SUB, ch2, n), x.dtype),
            pltpu.SemaphoreType.DMA((2, 2 * N_HOPS, N_SUB)),
            pltpu.SemaphoreType.DMA((2, 2 * N_HOPS, N_SUB)),
            pltpu.SemaphoreType.DMA((2, N_DEV)),
            pltpu.SemaphoreType.DMA((2, N_DEV, N_SUB)),
        ],
        compiler_params=pltpu.CompilerParams(
            collective_id=0,
            vmem_limit_bytes=63 * 1024 * 1024,
        ),
    )(x)


# device time: 302623 ns/iter; 1.0419x vs baseline; 1.0003x over previous
import contextlib
import os

import jax
import jax.numpy as jnp
from jax import lax
from jax.experimental import pallas as pl
from jax.experimental.pallas import tpu as pltpu

_PROFILE = os.environ.get("AR_PROFILE") == "1"


def _scope(name):
    return jax.named_scope(name) if _PROFILE else contextlib.nullcontext()


N_DEV = 4
N_HOPS = N_DEV - 1
N_SUB = 4
PLUS, MINUS = 0, 1


def kernel(x):
    m, n = x.shape
    half = m // 2
    ch = half // N_DEV
    ch2 = ch // N_SUB

    def body(
        x_ref, out_ref, acc_ref, comm_ref,
        send_sems, recv_sems, copy_sems, store_sems,
    ):
        my = lax.axis_index("i")
        left = (my + N_DEV - 1) % N_DEV
        right = (my + 1) % N_DEV

        peer_out = {PLUS: right, MINUS: left}
        peer_in = {PLUS: left, MINUS: right}

        def up(dirn, k):
            if dirn == PLUS:
                return (my - k + N_DEV) % N_DEV
            return (my + k) % N_DEV

        def rows(dirn, c, t):
            return pl.ds(dirn * half + c * ch + t * ch2, ch2)

        def load_chunk(dirn, k):
            c = up(dirn, k)
            rws = pl.ds(dirn * half + c * ch, ch)
            cp = pltpu.make_async_copy(
                x_ref.at[rws], acc_ref.at[rws], copy_sems.at[dirn, k]
            )
            cp.start()
            return cp

        loads = {
            (d, k): load_chunk(d, k)
            for k in range(N_DEV)
            for d in (PLUS, MINUS)
        }

        def store(dirn, k, t):
            rws = rows(dirn, up(dirn, k), t)
            cp = pltpu.make_async_copy(
                acc_ref.at[rws], out_ref.at[rws], store_sems.at[dirn, k, t]
            )
            cp.start()
            return cp

        with _scope("barrier"):
            barrier_sem = pltpu.get_barrier_semaphore()
            for nbr in (left, right):
                pl.semaphore_signal(
                    barrier_sem, inc=1,
                    device_id=(nbr,), device_id_type=pl.DeviceIdType.MESH,
                )
            pl.semaphore_wait(barrier_sem, 2)

        with _scope("load0"):
            loads[PLUS, 0].wait()
            loads[MINUS, 0].wait()

        def rchunk(dirn, h):
            return up(dirn, h + 1 if h < N_HOPS else h - N_HOPS)

        def schunk(dirn, h):
            return my if h == 0 else rchunk(dirn, h - 1)

        def mk(dirn, h, t, is_send):
            c = schunk(dirn, h) if is_send else rchunk(dirn, h)
            if h < N_HOPS:
                buf = comm_ref.at[dirn, h, t]
            else:
                buf = acc_ref.at[rows(dirn, c, t), :]
            return pltpu.make_async_remote_copy(
                src_ref=acc_ref.at[rows(dirn, c, t), :],
                dst_ref=buf,
                send_sem=send_sems.at[dirn, h, t],
                recv_sem=recv_sems.at[dirn, h, t],
                device_id=(peer_out[dirn] if is_send else peer_in[dirn],),
                device_id_type=pl.DeviceIdType.MESH,
            )

        with _scope("send0"):
            for t in range(N_SUB):
                for d in (PLUS, MINUS):
                    mk(d, 0, t, True).start()

        for h in range(2 * N_HOPS):
            with _scope(f"hop#h={h}"):
                if h < N_HOPS:
                    for d in (PLUS, MINUS):
                        loads[d, h + 1].wait()
                for t in range(N_SUB):
                    for d in (PLUS, MINUS):
                        mk(d, h, t, False).wait_recv()
                        if h < N_HOPS:
                            acc_ref[rows(d, rchunk(d, h), t), :] += (
                                comm_ref[d, h, t]
                            )
                            if h == N_HOPS - 1:
                                store(d, N_DEV - 1, t)
                        else:
                            store(d, h - N_HOPS, t)
                        if h + 1 < 2 * N_HOPS:
                            mk(d, h + 1, t, True).start()

        with _scope("drain"):
            for d in (PLUS, MINUS):
                for k in range(N_DEV):
                    for t in range(N_SUB):
                        rws = rows(d, up(d, k), t)
                        pltpu.make_async_copy(
                            acc_ref.at[rws], out_ref.at[rws],
                            store_sems.at[d, k, t],
                        ).wait()
            for h in range(2 * N_HOPS):
                for t in range(N_SUB):
                    for d in (PLUS, MINUS):
                        mk(d, h, t, True).wait_send()

    return pl.pallas_call(
        body,
        out_shape=jax.ShapeDtypeStruct((m, n), x.dtype),
        in_specs=[pl.BlockSpec(memory_space=pl.ANY)],
        out_specs=pl.BlockSpec(memory_space=pl.ANY),
        scratch_shapes=[
            pltpu.VMEM((m, n), x.dtype),
            pltpu.VMEM((2, N_HOPS, N_SUB, ch2, n), x.dtype),
            pltpu.SemaphoreType.DMA((2, 2 * N_HOPS, N_SUB)),
            pltpu.SemaphoreType.DMA((2, 2 * N_HOPS, N_SUB)),
            pltpu.SemaphoreType.DMA((2, N_DEV)),
            pltpu.SemaphoreType.DMA((2, N_DEV, N_SUB)),
        ],
        compiler_params=pltpu.CompilerParams(
            collective_id=0,
            vmem_limit_bytes=63 * 1024 * 1024,
        ),
    )(x)


# device time: 302557 ns/iter; 1.0421x vs baseline; 1.0002x over previous
import contextlib
import os

import jax
import jax.numpy as jnp
from jax import lax
from jax.experimental import pallas as pl
from jax.experimental.pallas import tpu as pltpu

_PROFILE = os.environ.get("AR_PROFILE") == "1"


def _scope(name):
    return jax.named_scope(name) if _PROFILE else contextlib.nullcontext()


N_DEV = 4
N_HOPS = N_DEV - 1
N_SUB = 2
PLUS, MINUS = 0, 1


def kernel(x):
    m, n = x.shape
    half = m // 2
    ch = half // N_DEV
    ch2 = ch // N_SUB

    def body(
        x_ref, out_ref, acc_ref, comm_ref,
        send_sems, recv_sems, copy_sems, store_sems,
    ):
        my = lax.axis_index("i")
        left = (my + N_DEV - 1) % N_DEV
        right = (my + 1) % N_DEV

        peer_out = {PLUS: right, MINUS: left}
        peer_in = {PLUS: left, MINUS: right}

        def up(dirn, k):
            if dirn == PLUS:
                return (my - k + N_DEV) % N_DEV
            return (my + k) % N_DEV

        def rows(dirn, c, t):
            return pl.ds(dirn * half + c * ch + t * ch2, ch2)

        def load_chunk(dirn, k):
            c = up(dirn, k)
            rws = pl.ds(dirn * half + c * ch, ch)
            cp = pltpu.make_async_copy(
                x_ref.at[rws], acc_ref.at[rws], copy_sems.at[dirn, k]
            )
            cp.start()
            return cp

        loads = {
            (d, k): load_chunk(d, k)
            for k in range(N_DEV)
            for d in (PLUS, MINUS)
        }

        def store(dirn, k, t):
            rws = rows(dirn, up(dirn, k), t)
            cp = pltpu.make_async_copy(
                acc_ref.at[rws], out_ref.at[rws], store_sems.at[dirn, k, t]
            )
            cp.start()
            return cp

        with _scope("barrier"):
            barrier_sem = pltpu.get_barrier_semaphore()
            for nbr in (left, right):
                pl.semaphore_signal(
                    barrier_sem, inc=1,
                    device_id=(nbr,), device_id_type=pl.DeviceIdType.MESH,
                )
            pl.semaphore_wait(barrier_sem, 2)

        with _scope("load0"):
            loads[PLUS, 0].wait()
            loads[MINUS, 0].wait()

        def rchunk(dirn, h):
            return up(dirn, h + 1 if h < N_HOPS else h - N_HOPS)

        def schunk(dirn, h):
            return my if h == 0 else rchunk(dirn, h - 1)

        def mk(dirn, h, t, is_send):
            c = schunk(dirn, h) if is_send else rchunk(dirn, h)
            if h < N_HOPS:
                buf = comm_ref.at[dirn, h, t]
            else:
                buf = acc_ref.at[rows(dirn, c, t), :]
            return pltpu.make_async_remote_copy(
                src_ref=acc_ref.at[rows(dirn, c, t), :],
                dst_ref=buf,
                send_sem=send_sems.at[dirn, h, t],
                recv_sem=recv_sems.at[dirn, h, t],
                device_id=(peer_out[dirn] if is_send else peer_in[dirn],),
                device_id_type=pl.DeviceIdType.MESH,
            )

        with _scope("send0"):
            for t in range(N_SUB):
                for d in (PLUS, MINUS):
                    mk(d, 0, t, True).start()

        for h in range(2 * N_HOPS):
            with _scope(f"hop#h={h}"):
                if h < N_HOPS:
                    for d in (PLUS, MINUS):
                        loads[d, h + 1].wait()
                for t in range(N_SUB):
                    for d in (PLUS, MINUS):
                        mk(d, h, t, False).wait_recv()
                        if h < N_HOPS:
                            acc_ref[rows(d, rchunk(d, h), t), :] += (
                                comm_ref[d, h, t]
                            )
                            if h == N_HOPS - 1:
                                store(d, N_DEV - 1, t)
                        else:
                            store(d, h - N_HOPS, t)
                        if h + 1 < 2 * N_HOPS:
                            mk(d, h + 1, t, True).start()

        with _scope("drain"):
            for d in (PLUS, MINUS):
                for k in range(N_DEV):
                    for t in range(N_SUB):
                        rws = rows(d, up(d, k), t)
                        pltpu.make_async_copy(
                            acc_ref.at[rws], out_ref.at[rws],
                            store_sems.at[d, k, t],
                        ).wait()
            for h in range(2 * N_HOPS):
                for t in range(N_SUB):
                    for d in (PLUS, MINUS):
                        mk(d, h, t, True).wait_send()

    return pl.pallas_call(
        body,
        out_shape=jax.ShapeDtypeStruct((m, n), x.dtype),
        in_specs=[pl.BlockSpec(memory_space=pl.ANY)],
        out_specs=pl.BlockSpec(memory_space=pl.ANY),
        scratch_shapes=[
            pltpu.VMEM((m, n), x.dtype),
            pltpu.VMEM((2, N_HOPS, N_SUB, ch2, n), x.dtype),
            pltpu.SemaphoreType.DMA((2, 2 * N_HOPS, N_SUB)),
            pltpu.SemaphoreType.DMA((2, 2 * N_HOPS, N_SUB)),
            pltpu.SemaphoreType.DMA((2, N_DEV)),
            pltpu.SemaphoreType.DMA((2, N_DEV, N_SUB)),
        ],
        compiler_params=pltpu.CompilerParams(
            collective_id=0,
            vmem_limit_bytes=63 * 1024 * 1024,
        ),
    )(x)


# device time: 301161 ns/iter; 1.0469x vs baseline; 1.0046x over previous
import contextlib
import os

import jax
import jax.numpy as jnp
from jax import lax
from jax.experimental import pallas as pl
from jax.experimental.pallas import tpu as pltpu

_PROFILE = os.environ.get("AR_PROFILE") == "1"


def _scope(name):
    return jax.named_scope(name) if _PROFILE else contextlib.nullcontext()


N_DEV = 4
N_HOPS = N_DEV - 1
N_SUB = 2
PLUS, MINUS = 0, 1


def kernel(x):
    m, n = x.shape
    half = m // 2
    ch = half // N_DEV
    ch2 = ch // N_SUB

    def body(
        x_ref, out_ref, acc_ref, comm_ref,
        send_sems, recv_sems, copy_sems, store_sems,
    ):
        my = lax.axis_index("i")
        left = (my + N_DEV - 1) % N_DEV
        right = (my + 1) % N_DEV

        peer_out = {PLUS: right, MINUS: left}
        peer_in = {PLUS: left, MINUS: right}

        def up(dirn, k):
            if dirn == PLUS:
                return (my - k + N_DEV) % N_DEV
            return (my + k) % N_DEV

        def rows(dirn, c, t):
            return pl.ds(dirn * half + c * ch + t * ch2, ch2)

        def load_chunk(dirn, k):
            c = up(dirn, k)
            rws = pl.ds(dirn * half + c * ch, ch)
            cp = pltpu.make_async_copy(
                x_ref.at[rws], acc_ref.at[rws], copy_sems.at[dirn, k]
            )
            cp.start()
            return cp

        loads = {
            (d, k): load_chunk(d, k)
            for k in range(1, N_DEV)
            for d in (PLUS, MINUS)
        }

        def store(dirn, k, t):
            rws = rows(dirn, up(dirn, k), t)
            cp = pltpu.make_async_copy(
                acc_ref.at[rws], out_ref.at[rws], store_sems.at[dirn, k, t]
            )
            cp.start()
            return cp

        with _scope("barrier"):
            barrier_sem = pltpu.get_barrier_semaphore()
            for nbr in (left, right):
                pl.semaphore_signal(
                    barrier_sem, inc=1,
                    device_id=(nbr,), device_id_type=pl.DeviceIdType.MESH,
                )
            pl.semaphore_wait(barrier_sem, 2)

        def rchunk(dirn, h):
            return up(dirn, h + 1 if h < N_HOPS else h - N_HOPS)

        def schunk(dirn, h):
            return my if h == 0 else rchunk(dirn, h - 1)

        def mk(dirn, h, t, is_send):
            c = schunk(dirn, h) if is_send else rchunk(dirn, h)
            if h < N_HOPS:
                buf = comm_ref.at[dirn, h, t]
            else:
                buf = acc_ref.at[rows(dirn, c, t), :]
            src = x_ref if is_send and h == 0 else acc_ref
            return pltpu.make_async_remote_copy(
                src_ref=src.at[rows(dirn, c, t), :],
                dst_ref=buf,
                send_sem=send_sems.at[dirn, h, t],
                recv_sem=recv_sems.at[dirn, h, t],
                device_id=(peer_out[dirn] if is_send else peer_in[dirn],),
                device_id_type=pl.DeviceIdType.MESH,
            )

        with _scope("send0"):
            for t in range(N_SUB):
                for d in (PLUS, MINUS):
                    mk(d, 0, t, True).start()

        for h in range(2 * N_HOPS):
            with _scope(f"hop#h={h}"):
                if h < N_HOPS:
                    for d in (PLUS, MINUS):
                        loads[d, h + 1].wait()
                for t in range(N_SUB):
                    for d in (PLUS, MINUS):
                        mk(d, h, t, False).wait_recv()
                        if h < N_HOPS:
                            acc_ref[rows(d, rchunk(d, h), t), :] += (
                                comm_ref[d, h, t]
                            )
                            if h == N_HOPS - 1:
                                store(d, N_DEV - 1, t)
                        else:
                            store(d, h - N_HOPS, t)
                        if h + 1 < 2 * N_HOPS:
                            mk(d, h + 1, t, True).start()

        with _scope("drain"):
            for d in (PLUS, MINUS):
                for k in range(N_DEV):
                    for t in range(N_SUB):
                        rws = rows(d, up(d, k), t)
                        pltpu.make_async_copy(
                            acc_ref.at[rws], out_ref.at[rws],
                            store_sems.at[d, k, t],
                        ).wait()
            for h in range(2 * N_HOPS):
                for t in range(N_SUB):
                    for d in (PLUS, MINUS):
                        mk(d, h, t, True).wait_send()

    return pl.pallas_call(
        body,
        out_shape=jax.ShapeDtypeStruct((m, n), x.dtype),
        in_specs=[pl.BlockSpec(memory_space=pl.ANY)],
        out_specs=pl.BlockSpec(memory_space=pl.ANY),
        scratch_shapes=[
            pltpu.VMEM((m, n), x.dtype),
            pltpu.VMEM((2, N_HOPS, N_SUB, ch2, n), x.dtype),
            pltpu.SemaphoreType.DMA((2, 2 * N_HOPS, N_SUB)),
            pltpu.SemaphoreType.DMA((2, 2 * N_HOPS, N_SUB)),
            pltpu.SemaphoreType.DMA((2, N_DEV)),
            pltpu.SemaphoreType.DMA((2, N_DEV, N_SUB)),
        ],
        compiler_params=pltpu.CompilerParams(
            collective_id=0,
            vmem_limit_bytes=63 * 1024 * 1024,
        ),
    )(x)
